# baseline (speedup 1.0000x reference)
import functools

import jax
import jax.numpy as jnp
from jax import lax
from jax.experimental import pallas as pl
from jax.experimental.pallas import tpu as pltpu
from jax.experimental.pallas import tpu_sc as plsc

NU = 50000
NI = 50000
NN = NU + NI
D = 32
H = 16
N_LAYERS = 3
E = 1600000
B = 4096
EPS = 1e-5
LOSS_W = 0.1

NC, NS = 2, 16
CHUNK = 1024
SUB = 512
NSUB = CHUNK // SUB
EPT = 100352
E_PAD = EPT * NS
NCHUNKS = EPT // CHUNK
NN_PAD = 100096
ROWS_PT = NN_PAD // NS
ZCOPIES = ROWS_PT // CHUNK
ZTAIL = ROWS_PT - ZCOPIES * CHUNK


def _seg_body(emb2, col2, row2, val, out,
              col_v, row_v, val_v, gath_v, acc, gsem, ssem):
    c = lax.axis_index("c")
    s = lax.axis_index("s")
    c2 = c

    @plsc.parallel_loop(0, CHUNK, unroll=4)
    def _z(i):
        gath_v[i, :] = jnp.zeros((H,), jnp.float32)

    for t in range(ZCOPIES):
        pltpu.sync_copy(gath_v, acc.at[pl.ds(s * ROWS_PT + t * CHUNK, CHUNK)])
    pltpu.sync_copy(gath_v.at[pl.ds(0, ZTAIL)],
                    acc.at[pl.ds(s * ROWS_PT + ZCOPIES * CHUNK, ZTAIL)])
    plsc.subcore_barrier()

    @pl.loop(0, NCHUNKS)
    def chunk_body(k):
        r0 = s * (EPT // SUB) + k * NSUB
        e0 = s * EPT + k * CHUNK
        pltpu.sync_copy(col2.at[pl.ds(r0, NSUB)], col_v)
        pltpu.sync_copy(row2.at[pl.ds(r0, NSUB)], row_v)
        pltpu.sync_copy(val.at[pl.ds(e0, CHUNK)], val_v)

        @pl.loop(0, NSUB)
        def _adjrow(j):
            @pl.loop(0, SUB // 16)
            def _adj(l):
                sl = pl.ds(l * 16, 16)
                col_v[j, sl] = col_v[j, sl] * 2 + c2

        gathers = []
        for j in range(NSUB):
            gathers.append(pltpu.async_copy(
                emb2.at[col_v.at[j]], gath_v.at[pl.ds(j * SUB, SUB)], gsem))
        for g in gathers:
            g.wait()

        @plsc.parallel_loop(0, CHUNK // 16)
        def _scale(g):
            wv = val_v[pl.ds(g * 16, 16)]
            e0g = g * 16
            for l in range(16):
                gath_v[e0g + l, :] = gath_v[e0g + l, :] * wv[l]

        scatters = []
        for j in range(NSUB):
            scatters.append(pltpu.async_copy(
                gath_v.at[pl.ds(j * SUB, SUB)], acc.at[row_v.at[j]], ssem,
                add=True))
        for sc in scatters:
            sc.wait()
    plsc.subcore_barrier()

    r0 = s * ROWS_PT
    pltpu.sync_copy(acc.at[pl.ds(r0, ROWS_PT)], out.at[c, pl.ds(r0, ROWS_PT)])


@functools.cache
def _seg_call():
    return pl.kernel(
        _seg_body,
        out_type=jax.ShapeDtypeStruct((NC, NN_PAD, H), jnp.float32),
        mesh=plsc.VectorSubcoreMesh(core_axis_name="c", subcore_axis_name="s",
                                    num_cores=NC, num_subcores=NS),
        scratch_types=[
            pltpu.VMEM((NSUB, SUB), jnp.int32),
            pltpu.VMEM((NSUB, SUB), jnp.int32),
            pltpu.VMEM((CHUNK,), jnp.float32),
            pltpu.VMEM((CHUNK, H), jnp.float32),
            pltpu.VMEM_SHARED((NN_PAD, H), jnp.float32),
            pltpu.SemaphoreType.DMA,
            pltpu.SemaphoreType.DMA,
        ],
        compiler_params=pltpu.CompilerParams(use_tc_tiling_on_sc=False),
    )


_B_PW = B // (NC * NS)


def _lookup_body(light, users, pos, neg, u_out, p_out, n_out,
                 idx_v, rows_v, sem):
    w = lax.axis_index("s") * NC + lax.axis_index("c")
    base = w * _B_PW

    for off, src, dst in ((0, users, u_out), (NU, pos, p_out), (NU, neg, n_out)):
        pltpu.sync_copy(src.at[pl.ds(base, _B_PW)], idx_v)
        if off:
            def addl(l, _):
                sl = pl.ds(l * 16, 16)
                idx_v[sl] = idx_v[sl] + off
                return 0
            lax.fori_loop(0, _B_PW // 16, addl, 0)
        pltpu.async_copy(light.at[idx_v], rows_v, sem).wait()
        pltpu.sync_copy(rows_v, dst.at[pl.ds(base, _B_PW)])


@functools.cache
def _lookup_call():
    return pl.kernel(
        _lookup_body,
        out_type=(jax.ShapeDtypeStruct((B, D), jnp.float32),) * 3,
        mesh=plsc.VectorSubcoreMesh(core_axis_name="c", subcore_axis_name="s",
                                    num_cores=NC, num_subcores=NS),
        scratch_types=[
            pltpu.VMEM((_B_PW,), jnp.int32),
            pltpu.VMEM((_B_PW, D), jnp.float32),
            pltpu.SemaphoreType.DMA,
        ],
        compiler_params=pltpu.CompilerParams(use_tc_tiling_on_sc=False),
    )


_RB = 2000


def _prep_body(cf_ref, wp_ref, bp_ref, wg_ref, bg_ref, it_ref,
               items_ref, proj_ref):
    cf = cf_ref[...]
    proj = jnp.dot(cf, wp_ref[...], preferred_element_type=jnp.float32,
                   precision=lax.Precision.HIGHEST) + bp_ref[...]
    logits = jnp.sum(cf * wg_ref[...], axis=1, keepdims=True) + bg_ref[0, 0]
    g = jax.nn.sigmoid(logits)
    items_ref[...] = (1.0 - g) * it_ref[...] + g * proj
    proj_ref[...] = proj


def _prep_call(content, W_proj, b_proj, W_gate, b_gate, item_table):
    return pl.pallas_call(
        _prep_body,
        grid=(NI // _RB,),
        in_specs=[
            pl.BlockSpec((_RB, 256), lambda i: (i, 0)),
            pl.BlockSpec((256, D), lambda i: (0, 0)),
            pl.BlockSpec((1, D), lambda i: (0, 0)),
            pl.BlockSpec((1, 256), lambda i: (0, 0)),
            pl.BlockSpec((1, 1), lambda i: (0, 0), memory_space=pltpu.SMEM),
            pl.BlockSpec((_RB, D), lambda i: (i, 0)),
        ],
        out_specs=[
            pl.BlockSpec((_RB, D), lambda i: (i, 0)),
            pl.BlockSpec((_RB, D), lambda i: (i, 0)),
        ],
        out_shape=[
            jax.ShapeDtypeStruct((NI, D), jnp.float32),
            jax.ShapeDtypeStruct((NI, D), jnp.float32),
        ],
    )(content, W_proj, b_proj.reshape(1, D), W_gate.reshape(1, 256),
      b_gate.reshape(1, 1), item_table)


def _ln_body(seg_ref, prev_ref, out_ref):
    x = jnp.concatenate([seg_ref[0], seg_ref[1]], axis=1)
    m = jnp.mean(x, axis=1, keepdims=True)
    d = x - m
    v = jnp.mean(d * d, axis=1, keepdims=True)
    out_ref[...] = d * lax.rsqrt(v + EPS) + prev_ref[...]


def _ln_call(seg, prev):
    return pl.pallas_call(
        _ln_body,
        grid=(NN // _RB,),
        in_specs=[
            pl.BlockSpec((NC, _RB, H), lambda i: (0, i, 0)),
            pl.BlockSpec((_RB, D), lambda i: (i, 0)),
        ],
        out_specs=pl.BlockSpec((_RB, D), lambda i: (i, 0)),
        out_shape=jax.ShapeDtypeStruct((NN, D), jnp.float32),
    )(seg, prev)


_NUB = NU // _RB


def _final_body(e0_ref, e1_ref, e2_ref, e3_ref, proj_ref, light_ref, loss_ref):
    i = pl.program_id(0)
    lt = (e0_ref[...] + e1_ref[...] + e2_ref[...] + e3_ref[...]) * 0.25
    light_ref[...] = lt

    @pl.when(i == 0)
    def _():
        loss_ref[0, 0] = 0.0

    @pl.when(i >= _NUB)
    def _():
        dd = lt - proj_ref[...]
        loss_ref[0, 0] += jnp.sum(dd * dd)


def _final_call(e0, e1, e2, e3, proj):
    nspec = pl.BlockSpec((_RB, D), lambda i: (i, 0))
    return pl.pallas_call(
        _final_body,
        grid=(NN // _RB,),
        in_specs=[nspec, nspec, nspec, nspec,
                  pl.BlockSpec((_RB, D), lambda i: (jnp.maximum(i - _NUB, 0), 0))],
        out_specs=[
            pl.BlockSpec((_RB, D), lambda i: (i, 0)),
            pl.BlockSpec((1, 1), lambda i: (0, 0), memory_space=pltpu.SMEM),
        ],
        out_shape=[
            jax.ShapeDtypeStruct((NN, D), jnp.float32),
            jax.ShapeDtypeStruct((1, 1), jnp.float32),
        ],
    )(e0, e1, e2, e3, proj)


def kernel(users, pos_items, neg_items, edge_index, graph_values,
           content_features, user_table, item_table, W_proj, b_proj,
           W_gate, b_gate):
    users = users.astype(jnp.int32)
    pos_items = pos_items.astype(jnp.int32)
    neg_items = neg_items.astype(jnp.int32)
    row = edge_index[0].astype(jnp.int32)
    col = edge_index[1].astype(jnp.int32)

    pad = E_PAD - E
    row2 = jnp.concatenate([row, jnp.full((pad,), NN, jnp.int32)])
    row2 = row2.reshape(E_PAD // SUB, SUB)
    col2 = jnp.concatenate([col, jnp.zeros((pad,), jnp.int32)])
    col2 = col2.reshape(E_PAD // SUB, SUB)
    val = jnp.concatenate([graph_values, jnp.zeros((pad,), jnp.float32)])

    items_emb, proj = _prep_call(content_features, W_proj, b_proj, W_gate,
                                 b_gate, item_table)
    emb = jnp.concatenate([user_table, items_emb], axis=0)

    embs = [emb]
    for _ in range(N_LAYERS):
        seg = _seg_call()(emb.reshape(2 * NN, H), col2, row2, val)
        emb = _ln_call(seg, emb)
        embs.append(emb)

    light, loss_sum = _final_call(embs[0], embs[1], embs[2], embs[3], proj)
    users_emb, pos_emb, neg_emb = _lookup_call()(light, users, pos_items,
                                                 neg_items)
    content_loss = loss_sum[0, 0] * (LOSS_W / (NI * D))
    return (users_emb, pos_emb, neg_emb, content_loss)


# speedup vs baseline: 1.4756x; 1.4756x over previous
import functools

import jax
import jax.numpy as jnp
from jax import lax
from jax.experimental import pallas as pl
from jax.experimental.pallas import tpu as pltpu
from jax.experimental.pallas import tpu_sc as plsc

NU = 50000
NI = 50000
NN = NU + NI
D = 32
H = 16
N_LAYERS = 3
E = 1600000
B = 4096
EPS = 1e-5
LOSS_W = 0.1

NC, NS = 2, 16
CHUNK = 512
SUB = 128
NSUB = CHUNK // SUB
EPT = 100352
E_PAD = EPT * NS
NCHUNKS = EPT // CHUNK
NIO = 4
NN_PAD = 100096
ROWS_PT = NN_PAD // NS
ZCOPIES = ROWS_PT // CHUNK
ZTAIL = ROWS_PT - ZCOPIES * CHUNK


def _seg_body(emb2, col2, row2, val, out,
              col_v, row_v, val_v, gath_v, acc, isem, gsemA, gsemB, ssem):
    c = lax.axis_index("c")
    s = lax.axis_index("s")
    c2 = c
    gsems = (gsemA, gsemB)

    @plsc.parallel_loop(0, CHUNK, unroll=4)
    def _z(i):
        gath_v[0, i, :] = jnp.zeros((H,), jnp.float32)

    for t in range(ZCOPIES):
        pltpu.sync_copy(gath_v.at[0],
                        acc.at[pl.ds(s * ROWS_PT + t * CHUNK, CHUNK)])
    pltpu.sync_copy(gath_v.at[0, pl.ds(0, ZTAIL)],
                    acc.at[pl.ds(s * ROWS_PT + ZCOPIES * CHUNK, ZTAIL)])
    plsc.subcore_barrier()

    def in_descs(m, q):
        r0 = s * (EPT // SUB) + m * NSUB
        e0 = s * EPT + m * CHUNK
        return ((col2.at[pl.ds(r0, NSUB)], col_v.at[q]),
                (row2.at[pl.ds(r0, NSUB)], row_v.at[q]),
                (val.at[pl.ds(e0, CHUNK)], val_v.at[q]))

    def fire_in(m, q):
        for src, dst in in_descs(m, q):
            pltpu.async_copy(src, dst, isem)

    def wait_in(m, q):
        for src, dst in in_descs(m, q):
            pltpu.make_async_copy(src, dst, isem).wait()

    def adjust(q):
        for j in range(NSUB):
            @plsc.parallel_loop(0, SUB // 16)
            def _adj(l):
                sl = pl.ds(l * 16, 16)
                col_v[q, j, sl] = col_v[q, j, sl] * 2 + c2

    def gather_descs(p, q):
        return tuple((emb2.at[col_v.at[q, j]],
                      gath_v.at[p, pl.ds(j * SUB, SUB)], gsems[p])
                     for j in range(NSUB))

    def scatter_descs(p, q):
        return tuple((gath_v.at[p, pl.ds(j * SUB, SUB)],
                      acc.at[row_v.at[q, j]]) for j in range(NSUB))

    def scale(p, q):
        @plsc.parallel_loop(0, CHUNK // 16)
        def _scale(g):
            wv = val_v[q, pl.ds(g * 16, 16)]
            e0g = g * 16
            for l in range(16):
                gath_v[p, e0g + l, :] = gath_v[p, e0g + l, :] * wv[l]

    def phase(k, p, q):
        @pl.when(k > 0)
        def _():
            for src, dst in scatter_descs(1 - p, (q - 1) % NIO):
                pltpu.make_async_copy(src, dst, ssem).wait()

        @pl.when(k < NCHUNKS - 1)
        def _():
            wait_in(k + 1, (q + 1) % NIO)
            adjust((q + 1) % NIO)
            for src, dst, sem in gather_descs(1 - p, (q + 1) % NIO):
                pltpu.async_copy(src, dst, sem)

        @pl.when(k < NCHUNKS - 2)
        def _():
            fire_in(k + 2, (q + 2) % NIO)

        for src, dst, sem in gather_descs(p, q):
            pltpu.make_async_copy(src, dst, sem).wait()
        scale(p, q)
        for src, dst in scatter_descs(p, q):
            pltpu.async_copy(src, dst, ssem, add=True)

    fire_in(0, 0)
    wait_in(0, 0)
    adjust(0)
    for src, dst, sem in gather_descs(0, 0):
        pltpu.async_copy(src, dst, sem)
    fire_in(1, 1)

    @pl.loop(0, NCHUNKS // NIO)
    def chunk_body(k6):
        k0 = k6 * NIO
        for ph in range(NIO):
            phase(k0 + ph, ph % 2, ph)

    for src, dst in scatter_descs((NCHUNKS - 1) % 2, (NCHUNKS - 1) % NIO):
        pltpu.make_async_copy(src, dst, ssem).wait()
    plsc.subcore_barrier()

    r0 = s * ROWS_PT
    pltpu.sync_copy(acc.at[pl.ds(r0, ROWS_PT)], out.at[c, pl.ds(r0, ROWS_PT)])


@functools.cache
def _seg_call():
    return pl.kernel(
        _seg_body,
        out_type=jax.ShapeDtypeStruct((NC, NN_PAD, H), jnp.float32),
        mesh=plsc.VectorSubcoreMesh(core_axis_name="c", subcore_axis_name="s",
                                    num_cores=NC, num_subcores=NS),
        scratch_types=[
            pltpu.VMEM((NIO, NSUB, SUB), jnp.int32),
            pltpu.VMEM((NIO, NSUB, SUB), jnp.int32),
            pltpu.VMEM((NIO, CHUNK), jnp.float32),
            pltpu.VMEM((2, CHUNK, H), jnp.float32),
            pltpu.VMEM_SHARED((NN_PAD, H), jnp.float32),
            pltpu.SemaphoreType.DMA,
            pltpu.SemaphoreType.DMA,
            pltpu.SemaphoreType.DMA,
            pltpu.SemaphoreType.DMA,
        ],
        compiler_params=pltpu.CompilerParams(use_tc_tiling_on_sc=False),
    )


_B_PW = B // (NC * NS)


def _lookup_body(light, users, pos, neg, u_out, p_out, n_out,
                 idx_v, rows_v, sem):
    w = lax.axis_index("s") * NC + lax.axis_index("c")
    base = w * _B_PW

    for off, src, dst in ((0, users, u_out), (NU, pos, p_out), (NU, neg, n_out)):
        pltpu.sync_copy(src.at[pl.ds(base, _B_PW)], idx_v)
        if off:
            def addl(l, _):
                sl = pl.ds(l * 16, 16)
                idx_v[sl] = idx_v[sl] + off
                return 0
            lax.fori_loop(0, _B_PW // 16, addl, 0)
        pltpu.async_copy(light.at[idx_v], rows_v, sem).wait()
        pltpu.sync_copy(rows_v, dst.at[pl.ds(base, _B_PW)])


@functools.cache
def _lookup_call():
    return pl.kernel(
        _lookup_body,
        out_type=(jax.ShapeDtypeStruct((B, D), jnp.float32),) * 3,
        mesh=plsc.VectorSubcoreMesh(core_axis_name="c", subcore_axis_name="s",
                                    num_cores=NC, num_subcores=NS),
        scratch_types=[
            pltpu.VMEM((_B_PW,), jnp.int32),
            pltpu.VMEM((_B_PW, D), jnp.float32),
            pltpu.SemaphoreType.DMA,
        ],
        compiler_params=pltpu.CompilerParams(use_tc_tiling_on_sc=False),
    )


_RB = 2000


def _prep_body(cf_ref, wp_ref, bp_ref, wg_ref, bg_ref, it_ref,
               items_ref, proj_ref):
    cf = cf_ref[...]
    proj = jnp.dot(cf, wp_ref[...], preferred_element_type=jnp.float32,
                   precision=lax.Precision.HIGHEST) + bp_ref[...]
    logits = jnp.sum(cf * wg_ref[...], axis=1, keepdims=True) + bg_ref[0, 0]
    g = jax.nn.sigmoid(logits)
    items_ref[...] = (1.0 - g) * it_ref[...] + g * proj
    proj_ref[...] = proj


def _prep_call(content, W_proj, b_proj, W_gate, b_gate, item_table):
    return pl.pallas_call(
        _prep_body,
        grid=(NI // _RB,),
        in_specs=[
            pl.BlockSpec((_RB, 256), lambda i: (i, 0)),
            pl.BlockSpec((256, D), lambda i: (0, 0)),
            pl.BlockSpec((1, D), lambda i: (0, 0)),
            pl.BlockSpec((1, 256), lambda i: (0, 0)),
            pl.BlockSpec((1, 1), lambda i: (0, 0), memory_space=pltpu.SMEM),
            pl.BlockSpec((_RB, D), lambda i: (i, 0)),
        ],
        out_specs=[
            pl.BlockSpec((_RB, D), lambda i: (i, 0)),
            pl.BlockSpec((_RB, D), lambda i: (i, 0)),
        ],
        out_shape=[
            jax.ShapeDtypeStruct((NI, D), jnp.float32),
            jax.ShapeDtypeStruct((NI, D), jnp.float32),
        ],
    )(content, W_proj, b_proj.reshape(1, D), W_gate.reshape(1, 256),
      b_gate.reshape(1, 1), item_table)


def _ln_body(seg_ref, prev_ref, out_ref):
    x = jnp.concatenate([seg_ref[0], seg_ref[1]], axis=1)
    m = jnp.mean(x, axis=1, keepdims=True)
    d = x - m
    v = jnp.mean(d * d, axis=1, keepdims=True)
    out_ref[...] = d * lax.rsqrt(v + EPS) + prev_ref[...]


def _ln_call(seg, prev):
    return pl.pallas_call(
        _ln_body,
        grid=(NN // _RB,),
        in_specs=[
            pl.BlockSpec((NC, _RB, H), lambda i: (0, i, 0)),
            pl.BlockSpec((_RB, D), lambda i: (i, 0)),
        ],
        out_specs=pl.BlockSpec((_RB, D), lambda i: (i, 0)),
        out_shape=jax.ShapeDtypeStruct((NN, D), jnp.float32),
    )(seg, prev)


_NUB = NU // _RB


def _final_body(e0_ref, e1_ref, e2_ref, e3_ref, proj_ref, light_ref, loss_ref):
    i = pl.program_id(0)
    lt = (e0_ref[...] + e1_ref[...] + e2_ref[...] + e3_ref[...]) * 0.25
    light_ref[...] = lt

    @pl.when(i == 0)
    def _():
        loss_ref[0, 0] = 0.0

    @pl.when(i >= _NUB)
    def _():
        dd = lt - proj_ref[...]
        loss_ref[0, 0] += jnp.sum(dd * dd)


def _final_call(e0, e1, e2, e3, proj):
    nspec = pl.BlockSpec((_RB, D), lambda i: (i, 0))
    return pl.pallas_call(
        _final_body,
        grid=(NN // _RB,),
        in_specs=[nspec, nspec, nspec, nspec,
                  pl.BlockSpec((_RB, D), lambda i: (jnp.maximum(i - _NUB, 0), 0))],
        out_specs=[
            pl.BlockSpec((_RB, D), lambda i: (i, 0)),
            pl.BlockSpec((1, 1), lambda i: (0, 0), memory_space=pltpu.SMEM),
        ],
        out_shape=[
            jax.ShapeDtypeStruct((NN, D), jnp.float32),
            jax.ShapeDtypeStruct((1, 1), jnp.float32),
        ],
    )(e0, e1, e2, e3, proj)


def kernel(users, pos_items, neg_items, edge_index, graph_values,
           content_features, user_table, item_table, W_proj, b_proj,
           W_gate, b_gate):
    users = users.astype(jnp.int32)
    pos_items = pos_items.astype(jnp.int32)
    neg_items = neg_items.astype(jnp.int32)
    row = edge_index[0].astype(jnp.int32)
    col = edge_index[1].astype(jnp.int32)

    pad = E_PAD - E
    row2 = jnp.concatenate([row, jnp.full((pad,), NN, jnp.int32)])
    row2 = row2.reshape(E_PAD // SUB, SUB)
    col2 = jnp.concatenate([col, jnp.zeros((pad,), jnp.int32)])
    col2 = col2.reshape(E_PAD // SUB, SUB)
    val = jnp.concatenate([graph_values, jnp.zeros((pad,), jnp.float32)])

    items_emb, proj = _prep_call(content_features, W_proj, b_proj, W_gate,
                                 b_gate, item_table)
    emb = jnp.concatenate([user_table, items_emb], axis=0)

    embs = [emb]
    for _ in range(N_LAYERS):
        seg = _seg_call()(emb.reshape(2 * NN, H), col2, row2, val)
        emb = _ln_call(seg, emb)
        embs.append(emb)

    light, loss_sum = _final_call(embs[0], embs[1], embs[2], embs[3], proj)
    users_emb, pos_emb, neg_emb = _lookup_call()(light, users, pos_items,
                                                 neg_items)
    content_loss = loss_sum[0, 0] * (LOSS_W / (NI * D))
    return (users_emb, pos_emb, neg_emb, content_loss)


# speedup vs baseline: 1.5215x; 1.0311x over previous
import functools

import jax
import jax.numpy as jnp
from jax import lax
from jax.experimental import pallas as pl
from jax.experimental.pallas import tpu as pltpu
from jax.experimental.pallas import tpu_sc as plsc

NU = 50000
NI = 50000
NN = NU + NI
D = 32
H = 16
N_LAYERS = 3
E = 1600000
B = 4096
EPS = 1e-5
LOSS_W = 0.1

NC, NS = 2, 16
CHUNK = 512
SUB = 128
NSUB = CHUNK // SUB
EPT = 100352
E_PAD = EPT * NS
NCHUNKS = EPT // CHUNK
NIO = 4
NN_PAD = 100096
ROWS_PT = NN_PAD // NS
ZCOPIES = ROWS_PT // CHUNK
ZTAIL = ROWS_PT - ZCOPIES * CHUNK


def _seg_body(emb2, col2, row2, val, out,
              col_v, row_v, val_v, gath_v, acc, isem, gsemA, gsemB, ssem):
    c = lax.axis_index("c")
    s = lax.axis_index("s")
    c2 = c
    gsems = (gsemA, gsemB)

    @plsc.parallel_loop(0, CHUNK, unroll=4)
    def _z(i):
        gath_v[0, i, :] = jnp.zeros((H,), jnp.float32)

    zcopies = [pltpu.async_copy(
        gath_v.at[0], acc.at[pl.ds(s * ROWS_PT + t * CHUNK, CHUNK)], isem)
        for t in range(ZCOPIES)]
    zcopies.append(pltpu.async_copy(
        gath_v.at[0, pl.ds(0, ZTAIL)],
        acc.at[pl.ds(s * ROWS_PT + ZCOPIES * CHUNK, ZTAIL)], isem))
    for z in zcopies:
        z.wait()
    plsc.subcore_barrier()

    def in_descs(m, q):
        r0 = s * (EPT // SUB) + m * NSUB
        e0 = s * EPT + m * CHUNK
        return ((col2.at[pl.ds(r0, NSUB)], col_v.at[q]),
                (row2.at[pl.ds(r0, NSUB)], row_v.at[q]),
                (val.at[pl.ds(e0, CHUNK)], val_v.at[q]))

    def fire_in(m, q):
        for src, dst in in_descs(m, q):
            pltpu.async_copy(src, dst, isem)

    def wait_in(m, q):
        for src, dst in in_descs(m, q):
            pltpu.make_async_copy(src, dst, isem).wait()

    def adjust(q):
        for j in range(NSUB):
            @plsc.parallel_loop(0, SUB // 16)
            def _adj(l):
                sl = pl.ds(l * 16, 16)
                col_v[q, j, sl] = col_v[q, j, sl] * 2 + c2

    def gather_descs(p, q):
        return tuple((emb2.at[col_v.at[q, j]],
                      gath_v.at[p, pl.ds(j * SUB, SUB)], gsems[p])
                     for j in range(NSUB))

    def scatter_descs(p, q):
        return tuple((gath_v.at[p, pl.ds(j * SUB, SUB)],
                      acc.at[row_v.at[q, j]]) for j in range(NSUB))

    def scale(p, q):
        @plsc.parallel_loop(0, CHUNK // 16)
        def _scale(g):
            wv = val_v[q, pl.ds(g * 16, 16)]
            e0g = g * 16
            for l in range(16):
                gath_v[p, e0g + l, :] = gath_v[p, e0g + l, :] * wv[l]

    def phase(k, p, q):
        @pl.when(k > 0)
        def _():
            for src, dst in scatter_descs(1 - p, (q - 1) % NIO):
                pltpu.make_async_copy(src, dst, ssem).wait()

        @pl.when(k < NCHUNKS - 1)
        def _():
            wait_in(k + 1, (q + 1) % NIO)
            adjust((q + 1) % NIO)
            for src, dst, sem in gather_descs(1 - p, (q + 1) % NIO):
                pltpu.async_copy(src, dst, sem)

        @pl.when(k < NCHUNKS - 2)
        def _():
            fire_in(k + 2, (q + 2) % NIO)

        for src, dst, sem in gather_descs(p, q):
            pltpu.make_async_copy(src, dst, sem).wait()
        scale(p, q)
        for src, dst in scatter_descs(p, q):
            pltpu.async_copy(src, dst, ssem, add=True)

    fire_in(0, 0)
    wait_in(0, 0)
    adjust(0)
    for src, dst, sem in gather_descs(0, 0):
        pltpu.async_copy(src, dst, sem)
    fire_in(1, 1)

    @pl.loop(0, NCHUNKS // NIO)
    def chunk_body(k6):
        k0 = k6 * NIO
        for ph in range(NIO):
            phase(k0 + ph, ph % 2, ph)

    for src, dst in scatter_descs((NCHUNKS - 1) % 2, (NCHUNKS - 1) % NIO):
        pltpu.make_async_copy(src, dst, ssem).wait()
    plsc.subcore_barrier()

    r0 = s * ROWS_PT
    pltpu.sync_copy(acc.at[pl.ds(r0, ROWS_PT)], out.at[c, pl.ds(r0, ROWS_PT)])


@functools.cache
def _seg_call():
    return pl.kernel(
        _seg_body,
        out_type=jax.ShapeDtypeStruct((NC, NN_PAD, H), jnp.float32),
        mesh=plsc.VectorSubcoreMesh(core_axis_name="c", subcore_axis_name="s",
                                    num_cores=NC, num_subcores=NS),
        scratch_types=[
            pltpu.VMEM((NIO, NSUB, SUB), jnp.int32),
            pltpu.VMEM((NIO, NSUB, SUB), jnp.int32),
            pltpu.VMEM((NIO, CHUNK), jnp.float32),
            pltpu.VMEM((2, CHUNK, H), jnp.float32),
            pltpu.VMEM_SHARED((NN_PAD, H), jnp.float32),
            pltpu.SemaphoreType.DMA,
            pltpu.SemaphoreType.DMA,
            pltpu.SemaphoreType.DMA,
            pltpu.SemaphoreType.DMA,
        ],
        compiler_params=pltpu.CompilerParams(use_tc_tiling_on_sc=False),
    )


_B_PW = B // (NC * NS)


def _lookup_body(e0, e1, e2, e3, users, pos, neg, u_out, p_out, n_out,
                 idx_v, r0_v, r1_v, r2_v, r3_v, sem):
    w = lax.axis_index("s") * NC + lax.axis_index("c")
    base = w * _B_PW
    rbufs = (r0_v, r1_v, r2_v, r3_v)

    for off, src, dst in ((0, users, u_out), (NU, pos, p_out), (NU, neg, n_out)):
        pltpu.sync_copy(src.at[pl.ds(base, _B_PW)], idx_v)
        if off:
            @plsc.parallel_loop(0, _B_PW // 16)
            def _addl(l):
                sl = pl.ds(l * 16, 16)
                idx_v[sl] = idx_v[sl] + off
        copies = [pltpu.async_copy(t.at[idx_v], r, sem)
                  for t, r in zip((e0, e1, e2, e3), rbufs)]
        for cp in copies:
            cp.wait()

        @plsc.parallel_loop(0, _B_PW * (D // 16))
        def _avg(i):
            r = i // (D // 16)
            dcol = (i % (D // 16)) * 16
            sl = pl.ds(dcol, 16)
            r0_v[r, sl] = (r0_v[r, sl] + r1_v[r, sl] +
                           r2_v[r, sl] + r3_v[r, sl]) * 0.25

        pltpu.sync_copy(r0_v, dst.at[pl.ds(base, _B_PW)])


@functools.cache
def _lookup_call():
    return pl.kernel(
        _lookup_body,
        out_type=(jax.ShapeDtypeStruct((B, D), jnp.float32),) * 3,
        mesh=plsc.VectorSubcoreMesh(core_axis_name="c", subcore_axis_name="s",
                                    num_cores=NC, num_subcores=NS),
        scratch_types=[
            pltpu.VMEM((_B_PW,), jnp.int32),
            pltpu.VMEM((_B_PW, D), jnp.float32),
            pltpu.VMEM((_B_PW, D), jnp.float32),
            pltpu.VMEM((_B_PW, D), jnp.float32),
            pltpu.VMEM((_B_PW, D), jnp.float32),
            pltpu.SemaphoreType.DMA,
        ],
        compiler_params=pltpu.CompilerParams(use_tc_tiling_on_sc=False),
    )


_RB = 2000


def _prep_body(cf_ref, wp_ref, bp_ref, wg_ref, bg_ref, it_ref,
               items_ref, proj_ref):
    cf = cf_ref[...]
    proj = jnp.dot(cf, wp_ref[...], preferred_element_type=jnp.float32,
                   precision=lax.Precision.HIGHEST) + bp_ref[...]
    logits = jnp.sum(cf * wg_ref[...], axis=1, keepdims=True) + bg_ref[0, 0]
    g = jax.nn.sigmoid(logits)
    items_ref[...] = (1.0 - g) * it_ref[...] + g * proj
    proj_ref[...] = proj


def _prep_call(content, W_proj, b_proj, W_gate, b_gate, item_table):
    return pl.pallas_call(
        _prep_body,
        grid=(NI // _RB,),
        in_specs=[
            pl.BlockSpec((_RB, 256), lambda i: (i, 0)),
            pl.BlockSpec((256, D), lambda i: (0, 0)),
            pl.BlockSpec((1, D), lambda i: (0, 0)),
            pl.BlockSpec((1, 256), lambda i: (0, 0)),
            pl.BlockSpec((1, 1), lambda i: (0, 0), memory_space=pltpu.SMEM),
            pl.BlockSpec((_RB, D), lambda i: (i, 0)),
        ],
        out_specs=[
            pl.BlockSpec((_RB, D), lambda i: (i, 0)),
            pl.BlockSpec((_RB, D), lambda i: (i, 0)),
        ],
        out_shape=[
            jax.ShapeDtypeStruct((NI, D), jnp.float32),
            jax.ShapeDtypeStruct((NI, D), jnp.float32),
        ],
    )(content, W_proj, b_proj.reshape(1, D), W_gate.reshape(1, 256),
      b_gate.reshape(1, 1), item_table)


def _ln_body(seg_ref, prev_ref, out_ref):
    x = jnp.concatenate([seg_ref[0], seg_ref[1]], axis=1)
    m = jnp.mean(x, axis=1, keepdims=True)
    d = x - m
    v = jnp.mean(d * d, axis=1, keepdims=True)
    out_ref[...] = d * lax.rsqrt(v + EPS) + prev_ref[...]


def _ln_call(seg, prev):
    return pl.pallas_call(
        _ln_body,
        grid=(NN // _RB,),
        in_specs=[
            pl.BlockSpec((NC, _RB, H), lambda i: (0, i, 0)),
            pl.BlockSpec((_RB, D), lambda i: (i, 0)),
        ],
        out_specs=pl.BlockSpec((_RB, D), lambda i: (i, 0)),
        out_shape=jax.ShapeDtypeStruct((NN, D), jnp.float32),
    )(seg, prev)


_NUB = NU // _RB


def _final_body(e0_ref, e1_ref, e2_ref, e3_ref, proj_ref, loss_ref):
    i = pl.program_id(0)
    lt = (e0_ref[...] + e1_ref[...] + e2_ref[...] + e3_ref[...]) * 0.25
    dd = lt - proj_ref[...]

    @pl.when(i == 0)
    def _():
        loss_ref[0, 0] = 0.0

    loss_ref[0, 0] += jnp.sum(dd * dd)


def _final_call(e0, e1, e2, e3, proj):
    ispec = pl.BlockSpec((_RB, D), lambda i: (i + _NUB, 0))
    return pl.pallas_call(
        _final_body,
        grid=(NI // _RB,),
        in_specs=[ispec, ispec, ispec, ispec,
                  pl.BlockSpec((_RB, D), lambda i: (i, 0))],
        out_specs=pl.BlockSpec((1, 1), lambda i: (0, 0),
                               memory_space=pltpu.SMEM),
        out_shape=jax.ShapeDtypeStruct((1, 1), jnp.float32),
    )(e0, e1, e2, e3, proj)


def kernel(users, pos_items, neg_items, edge_index, graph_values,
           content_features, user_table, item_table, W_proj, b_proj,
           W_gate, b_gate):
    users = users.astype(jnp.int32)
    pos_items = pos_items.astype(jnp.int32)
    neg_items = neg_items.astype(jnp.int32)
    row = edge_index[0].astype(jnp.int32)
    col = edge_index[1].astype(jnp.int32)

    pad = E_PAD - E
    row2 = jnp.concatenate([row, jnp.full((pad,), NN, jnp.int32)])
    row2 = row2.reshape(E_PAD // SUB, SUB)
    col2 = jnp.concatenate([col, jnp.zeros((pad,), jnp.int32)])
    col2 = col2.reshape(E_PAD // SUB, SUB)
    val = jnp.concatenate([graph_values, jnp.zeros((pad,), jnp.float32)])

    items_emb, proj = _prep_call(content_features, W_proj, b_proj, W_gate,
                                 b_gate, item_table)
    emb = jnp.concatenate([user_table, items_emb], axis=0)

    embs = [emb]
    for _ in range(N_LAYERS):
        seg = _seg_call()(emb.reshape(2 * NN, H), col2, row2, val)
        emb = _ln_call(seg, emb)
        embs.append(emb)

    loss_sum = _final_call(embs[0], embs[1], embs[2], embs[3], proj)
    users_emb, pos_emb, neg_emb = _lookup_call()(
        embs[0], embs[1], embs[2], embs[3], users, pos_items, neg_items)
    content_loss = loss_sum[0, 0] * (LOSS_W / (NI * D))
    return (users_emb, pos_emb, neg_emb, content_loss)
